# baseline (speedup 1.0000x reference)
import jax
import jax.numpy as jnp
from jax import lax
from jax.experimental import pallas as pl
from jax.experimental.pallas import tpu as pltpu
from jax.experimental.pallas import tpu_sc as plsc

NUM_EMB = 8192
DIM = 128
B = 64
HW = 1024
N = B * HW

NC = 2
NS = 16
NW = NC * NS

PER_TILE = N // NW
CHUNK = 128
NCHUNK = PER_TILE // CHUNK
NBUF = 3
HB = 6


def _body(q_hbm, w_hbm, out_hbm, idxbuf, rows, shared_w, gsems, ssems, stsem):
    c = lax.axis_index("c")
    s = lax.axis_index("s")
    wid = s * NC + c
    base = wid * PER_TILE

    ROWS_STAGE = NUM_EMB // NS
    sl = pl.ds(s * ROWS_STAGE, ROWS_STAGE)
    pltpu.async_copy(w_hbm.at[sl, :], shared_w.at[sl, :], stsem)

    pltpu.sync_copy(q_hbm.at[pl.ds(base, PER_TILE)], idxbuf)

    def start_gather(ch, p):
        idx = idxbuf.at[pl.ds(ch * CHUNK, CHUNK)]
        pltpu.async_copy(shared_w.at[idx], rows.at[p], gsems.at[p])

    def start_gather_hbm(ch, p):
        idx = idxbuf.at[pl.ds(ch * CHUNK, CHUNK)]
        pltpu.async_copy(w_hbm.at[idx], rows.at[p], gsems.at[p])

    def wait_gather(ch, p):
        idx = idxbuf.at[pl.ds(ch * CHUNK, CHUNK)]
        pltpu.make_async_copy(shared_w.at[idx], rows.at[p], gsems.at[p]).wait()

    def out_slice(ch):
        return out_hbm.at[pl.ds(base + ch * CHUNK, CHUNK), :]

    def start_store(ch, p):
        pltpu.async_copy(rows.at[p], out_slice(ch), ssems.at[p])

    def wait_store(ch, p):
        pltpu.make_async_copy(rows.at[p], out_slice(ch), ssems.at[p]).wait()

    for ch in range(NBUF):
        start_gather_hbm(ch, ch)
    for ch in range(HB):
        p = ch % NBUF
        wait_gather(ch, p)
        start_store(ch, p)
        if ch + NBUF < HB:
            wait_store(ch, p)
            start_gather_hbm(ch + NBUF, p)

    pltpu.make_async_copy(w_hbm.at[sl, :], shared_w.at[sl, :], stsem).wait()
    plsc.subcore_barrier()

    for ch in range(HB, HB + NBUF):
        p = ch % NBUF
        wait_store(ch - NBUF, p)
        start_gather(ch, p)

    DELAY = 2

    def chunk_body(ch, carry):
        p = lax.rem(ch, NBUF)
        wait_gather(ch, p)
        start_store(ch, p)
        d = ch - DELAY

        @pl.when(d >= HB)
        def _():
            dp = lax.rem(d, NBUF)
            wait_store(d, dp)
            start_gather(d + NBUF, dp)

        return carry

    lax.fori_loop(HB, NCHUNK - NBUF + DELAY, chunk_body, 0)
    for ch in range(NCHUNK - NBUF + DELAY, NCHUNK):
        p = ch % NBUF
        wait_gather(ch, p)
        start_store(ch, p)
    for ch in range(NCHUNK - NBUF, NCHUNK):
        wait_store(ch, ch % NBUF)


@jax.jit
def _lookup(q_flat, w):
    mesh = plsc.VectorSubcoreMesh(core_axis_name="c", subcore_axis_name="s")
    f = pl.kernel(
        _body,
        out_type=jax.ShapeDtypeStruct((N, DIM), jnp.float32),
        mesh=mesh,
        scratch_types=[
            pltpu.VMEM((PER_TILE,), jnp.int32),
            pltpu.VMEM((NBUF, CHUNK, DIM), jnp.float32),
            pltpu.VMEM_SHARED((NUM_EMB, DIM), jnp.float32),
            pltpu.SemaphoreType.DMA((NBUF,)),
            pltpu.SemaphoreType.DMA((NBUF,)),
            pltpu.SemaphoreType.DMA,
        ],
        compiler_params=pltpu.CompilerParams(
            use_tc_tiling_on_sc=False, needs_layout_passes=False
        ),
    )
    return f(q_flat, w)


def kernel(quantized, embedding_weight):
    q_flat = quantized.reshape(N)
    rows = _lookup(q_flat, embedding_weight)
    emb = rows.reshape(B, 32, 32, DIM).transpose(0, 3, 1, 2)
    return (quantized, emb)
